# baseline (speedup 1.0000x reference)
import functools

import jax
import jax.numpy as jnp
from jax import lax
from jax.experimental import pallas as pl
from jax.experimental.pallas import tpu as pltpu
from jax.experimental.pallas import tpu_sc as plsc

N = 10000
NP = 10240
E = 320000
NC = 2
NS = 16
NW = NC * NS
CHUNK = 128
TOTCH = 158
NCH0 = 128
NCH1 = TOTCH - NCH0
EP = NS * TOTCH * CHUNK
DNCH = EP // (NW * CHUNK)
RPT = NP // NS
RCOPY = RPT // CHUNK
NROW = NP // 128

_mesh = plsc.VectorSubcoreMesh(core_axis_name="c", subcore_axis_name="s")


def _zero_vmem(ref, nrow, ncol):
    def zrow(i, _):
        def zcol(j, _):
            ref[i, pl.ds(j * 16, 16)] = jnp.zeros((16,), jnp.float32)
            return 0
        return lax.fori_loop(0, ncol // 16, zcol, 0)
    lax.fori_loop(0, nrow, zrow, 0)


def _deg_body(srcr, dstr, dego, degi, src_v, dst_v, hist_o, hist_i,
              red_in, red_out, ho_sh, hi_sh):
    c = lax.axis_index("c")
    s = lax.axis_index("s")
    pltpu.sync_copy(srcr.at[c, s], src_v)
    pltpu.sync_copy(dstr.at[c, s], dst_v)
    _zero_vmem(hist_o, NROW, 128)
    _zero_vmem(hist_i, NROW, 128)
    ones = jnp.ones((16,), jnp.float32)

    def body(j, _):
        def inner(k, _):
            si = src_v[j, pl.ds(k * 16, 16)]
            plsc.addupdate_scatter(
                hist_o,
                [lax.shift_right_logical(si, 7), jnp.bitwise_and(si, 127)],
                ones)
            di = dst_v[j, pl.ds(k * 16, 16)]
            plsc.addupdate_scatter(
                hist_i,
                [lax.shift_right_logical(di, 7), jnp.bitwise_and(di, 127)],
                ones)
            return 0
        return lax.fori_loop(0, 8, inner, 0)
    lax.fori_loop(0, DNCH, body, 0)

    pltpu.sync_copy(hist_o, ho_sh.at[s])
    pltpu.sync_copy(hist_i, hi_sh.at[s])
    plsc.subcore_barrier()

    @pl.when(s < NROW // 8)
    def _():
        rb = pl.multiple_of(8 * s, 8)

        def _reduce(sh, out_hbm):
            for r in range(NS):
                pltpu.sync_copy(sh.at[r, pl.ds(rb, 8)], red_in.at[r])

            def ri(i, _):
                def rj(jj, _):
                    acc = red_in[0, i, pl.ds(jj * 16, 16)]
                    for r in range(1, NS):
                        acc = acc + red_in[r, i, pl.ds(jj * 16, 16)]
                    red_out[i, pl.ds(jj * 16, 16)] = acc
                    return 0
                return lax.fori_loop(0, 8, rj, 0)
            lax.fori_loop(0, 8, ri, 0)
            pltpu.sync_copy(red_out, out_hbm.at[c, pl.ds(rb, 8)])

        _reduce(ho_sh, dego)
        _reduce(hi_sh, degi)


_deg_call = functools.partial(
    pl.kernel,
    out_type=(
        jax.ShapeDtypeStruct((NC, NROW, 128), jnp.float32),
        jax.ShapeDtypeStruct((NC, NROW, 128), jnp.float32),
    ),
    mesh=_mesh,
    scratch_types=[
        pltpu.VMEM((DNCH, CHUNK), jnp.int32),
        pltpu.VMEM((DNCH, CHUNK), jnp.int32),
        pltpu.VMEM((NROW, 128), jnp.float32),
        pltpu.VMEM((NROW, 128), jnp.float32),
        pltpu.VMEM((NS, 8, 128), jnp.float32),
        pltpu.VMEM((8, 128), jnp.float32),
        pltpu.VMEM_SHARED((NS, NROW, 128), jnp.float32),
        pltpu.VMEM_SHARED((NS, NROW, 128), jnp.float32),
    ],
    compiler_params=pltpu.CompilerParams(needs_layout_passes=False),
)(_deg_body)


NBUF = 2
WND = 40


def _scatter_body(t_hbm, srcr, dstr, out, src_v, dst_v, rows_v, acc_sh,
                  *sems):
    gsem = sems[:NBUF]
    ssem = sems[NBUF:]
    D = rows_v.shape[2]
    c = lax.axis_index("c")
    s = lax.axis_index("s")

    _zero_vmem(rows_v.at[0], CHUNK, D)
    base = pl.multiple_of(s * RPT, RPT)
    for k in range(RCOPY):
        pltpu.sync_copy(rows_v.at[0], acc_sh.at[pl.ds(base + k * CHUNK, CHUNK)])
    plsc.subcore_barrier()

    def _wait_gather(b):
        pltpu.make_async_copy(
            t_hbm.at[src_v.at[0]], rows_v.at[b], gsem[b]).wait()

    def _wait_scatter(b):
        pltpu.make_async_copy(
            rows_v.at[b], acc_sh.at[dst_v.at[0]], ssem[b]).wait()

    def run(chunk0, nchunk):
        off = 0
        while off < nchunk:
            w = min(WND, nchunk - off)
            pltpu.sync_copy(srcr.at[s, pl.ds(chunk0 + off, w)],
                            src_v.at[pl.ds(0, w)])
            pltpu.sync_copy(dstr.at[s, pl.ds(chunk0 + off, w)],
                            dst_v.at[pl.ds(0, w)])
            for b in range(NBUF):
                pltpu.async_copy(t_hbm.at[src_v.at[b]], rows_v.at[b], gsem[b])

            def outer(g, _):
                j0 = g * NBUF
                for b in range(NBUF):
                    _wait_gather(b)
                    pltpu.async_copy(rows_v.at[b], acc_sh.at[dst_v.at[j0 + b]],
                                     ssem[b], add=True)
                for b in range(NBUF):
                    _wait_scatter(b)

                    @pl.when(j0 + NBUF + b < w)
                    def _():
                        pltpu.async_copy(t_hbm.at[src_v.at[j0 + NBUF + b]],
                                         rows_v.at[b], gsem[b])
                return 0
            lax.fori_loop(0, w // NBUF, outer, 0)
            off += w

    @pl.when(c == 0)
    def _():
        run(0, NCH0)

    @pl.when(c == 1)
    def _():
        run(NCH0, NCH1)

    plsc.subcore_barrier()
    pltpu.sync_copy(acc_sh.at[pl.ds(base, RPT)], out.at[c, pl.ds(base, RPT)])


def _make_scatter(D):
    return functools.partial(
        pl.kernel,
        out_type=jax.ShapeDtypeStruct((NC, NP, D), jnp.float32),
        mesh=_mesh,
        scratch_types=[
            pltpu.VMEM((WND, CHUNK), jnp.int32),
            pltpu.VMEM((WND, CHUNK), jnp.int32),
            pltpu.VMEM((NBUF, CHUNK, D), jnp.float32),
            pltpu.VMEM_SHARED((NP, D), jnp.float32),
        ] + [pltpu.SemaphoreType.DMA] * (2 * NBUF),
        compiler_params=pltpu.CompilerParams(needs_layout_passes=False),
    )(_scatter_body)


_scatter_128 = _make_scatter(128)


_RT = 1024
_RF = 1000


def _norm_from(deg2):
    d = deg2[0] + deg2[1]
    return jnp.where(d > 0, lax.rsqrt(jnp.maximum(d, 1.0)), 0.0)


def _tcmm_body(x_ref, w_ref, t_ref):
    t_ref[...] = jnp.dot(x_ref[...], w_ref[...],
                         preferred_element_type=jnp.float32)


def _tcscale_body(t_ref, dego_ref, o_ref):
    o_ref[...] = t_ref[...] * _norm_from(dego_ref[...])


def _tc_mid_body(acc_ref, dego_ref, degi_ref, b_ref, w_ref, t_ref):
    nd = _norm_from(degi_ref[...])
    agg = (acc_ref[0] + acc_ref[1]) * nd + b_ref[...]
    h = jnp.maximum(agg, 0.0)
    ns = _norm_from(dego_ref[...])
    t_ref[...] = jnp.dot(h * ns, w_ref[...], preferred_element_type=jnp.float32)


def _tc_fin_body(acc_ref, degi_ref, b_ref, o_ref):
    nd = _norm_from(degi_ref[...])
    agg = (acc_ref[0] + acc_ref[1])[:, :64] * nd + b_ref[...]
    o_ref[...] = jax.nn.sigmoid(agg) + 1e-8


def _deg_spec():
    return pl.BlockSpec((NC, _RT, 1), lambda i: (0, i, 0))


def _tcmm(x, w):
    dout = w.shape[1]
    return pl.pallas_call(
        _tcmm_body,
        grid=(N // _RF,),
        in_specs=[
            pl.BlockSpec((_RF, x.shape[1]), lambda i: (i, 0)),
            pl.BlockSpec(w.shape, lambda i: (0, 0)),
        ],
        out_specs=pl.BlockSpec((_RF, dout), lambda i: (i, 0)),
        out_shape=jax.ShapeDtypeStruct((NP, dout), jnp.float32),
    )(x, w)


def _tcscale(t, dego):
    dout = t.shape[1]
    return pl.pallas_call(
        _tcscale_body,
        grid=(NP // _RT,),
        in_specs=[
            pl.BlockSpec((_RT, dout), lambda i: (i, 0)),
            _deg_spec(),
        ],
        out_specs=pl.BlockSpec((_RT, dout), lambda i: (i, 0)),
        out_shape=jax.ShapeDtypeStruct((NP, dout), jnp.float32),
    )(t, dego)


def _tc_mid(acc, dego, degi, b, w):
    din, dout = w.shape
    return pl.pallas_call(
        _tc_mid_body,
        grid=(NP // _RT,),
        in_specs=[
            pl.BlockSpec((NC, _RT, din), lambda i: (0, i, 0)),
            _deg_spec(),
            _deg_spec(),
            pl.BlockSpec((1, din), lambda i: (0, 0)),
            pl.BlockSpec(w.shape, lambda i: (0, 0)),
        ],
        out_specs=pl.BlockSpec((_RT, dout), lambda i: (i, 0)),
        out_shape=jax.ShapeDtypeStruct((NP, dout), jnp.float32),
    )(acc, dego, degi, b, w)


def _tc_fin(acc, degi, b):
    return pl.pallas_call(
        _tc_fin_body,
        grid=(NP // _RT,),
        in_specs=[
            pl.BlockSpec((NC, _RT, 128), lambda i: (0, i, 0)),
            _deg_spec(),
            pl.BlockSpec((1, 64), lambda i: (0, 0)),
        ],
        out_specs=pl.BlockSpec((_RT, 64), lambda i: (i, 0)),
        out_shape=jax.ShapeDtypeStruct((N, 64), jnp.float32),
    )(acc, degi, b)


def kernel(features, edge_index, W0, b0, W1, b1, W2, b2):
    pad = EP - E
    ep = jnp.concatenate(
        [edge_index, jnp.full((2, pad), N, dtype=jnp.int32)], axis=1)
    src = ep[0].reshape(NS, TOTCH, CHUNK)
    dst = ep[1].reshape(NS, TOTCH, CHUNK)
    src_dg = ep[0].reshape(NC, NS, DNCH, CHUNK)
    dst_dg = ep[1].reshape(NC, NS, DNCH, CHUNK)
    dego_g, degi_g = _deg_call(src_dg, dst_dg)
    dego = dego_g.reshape(NC, NP, 1)
    degi = degi_g.reshape(NC, NP, 1)
    t0r = _tcmm(features, W0)
    t0 = _tcscale(t0r, dego)
    a0 = _scatter_128(t0, src, dst)
    t1 = _tc_mid(a0, dego, degi, b0.reshape(1, -1), W1)
    a1 = _scatter_128(t1, src, dst)
    W2p = jnp.pad(W2, ((0, 0), (0, 64)))
    t2 = _tc_mid(a1, dego, degi, b1.reshape(1, -1), W2p)
    a2 = _scatter_128(t2, src, dst)
    return _tc_fin(a2, degi, b2.reshape(1, -1))


# speedup vs baseline: 1.0100x; 1.0100x over previous
import functools

import jax
import jax.numpy as jnp
from jax import lax
from jax.experimental import pallas as pl
from jax.experimental.pallas import tpu as pltpu
from jax.experimental.pallas import tpu_sc as plsc

N = 10000
NP = 10112
NPD = 10240
E = 320000
NC = 2
NS = 16
NW = NC * NS
CHUNK = 128
TOTCH = 158
NCH0 = 128
NCH1 = TOTCH - NCH0
EP = NS * TOTCH * CHUNK
DNCH = EP // (NW * CHUNK)
RPT = NP // NS
NROW = NPD // 128

_mesh = plsc.VectorSubcoreMesh(core_axis_name="c", subcore_axis_name="s")


def _zero_vmem(ref, nrow, ncol):
    def zrow(i, _):
        def zcol(j, _):
            ref[i, pl.ds(j * 16, 16)] = jnp.zeros((16,), jnp.float32)
            return 0
        return lax.fori_loop(0, ncol // 16, zcol, 0)
    lax.fori_loop(0, nrow, zrow, 0)


def _deg_body(srcr, dstr, dego, degi, src_v, dst_v, hist_o, hist_i,
              red_in, red_out, ho_sh, hi_sh):
    c = lax.axis_index("c")
    s = lax.axis_index("s")
    pltpu.sync_copy(srcr.at[c, s], src_v)
    pltpu.sync_copy(dstr.at[c, s], dst_v)
    _zero_vmem(hist_o, NROW, 128)
    _zero_vmem(hist_i, NROW, 128)
    ones = jnp.ones((16,), jnp.float32)

    def body(j, _):
        def inner(k, _):
            si = src_v[j, pl.ds(k * 16, 16)]
            plsc.addupdate_scatter(
                hist_o,
                [lax.shift_right_logical(si, 7), jnp.bitwise_and(si, 127)],
                ones)
            di = dst_v[j, pl.ds(k * 16, 16)]
            plsc.addupdate_scatter(
                hist_i,
                [lax.shift_right_logical(di, 7), jnp.bitwise_and(di, 127)],
                ones)
            return 0
        return lax.fori_loop(0, 8, inner, 0)
    lax.fori_loop(0, DNCH, body, 0)

    pltpu.sync_copy(hist_o, ho_sh.at[s])
    pltpu.sync_copy(hist_i, hi_sh.at[s])
    plsc.subcore_barrier()

    @pl.when(s < NROW // 8)
    def _():
        rb = pl.multiple_of(8 * s, 8)

        def _reduce(sh, out_hbm):
            for r in range(NS):
                pltpu.sync_copy(sh.at[r, pl.ds(rb, 8)], red_in.at[r])

            def ri(i, _):
                def rj(jj, _):
                    acc = red_in[0, i, pl.ds(jj * 16, 16)]
                    for r in range(1, NS):
                        acc = acc + red_in[r, i, pl.ds(jj * 16, 16)]
                    red_out[i, pl.ds(jj * 16, 16)] = acc
                    return 0
                return lax.fori_loop(0, 8, rj, 0)
            lax.fori_loop(0, 8, ri, 0)
            pltpu.sync_copy(red_out, out_hbm.at[c, pl.ds(rb, 8)])

        _reduce(ho_sh, dego)
        _reduce(hi_sh, degi)


_deg_call = functools.partial(
    pl.kernel,
    out_type=(
        jax.ShapeDtypeStruct((NC, NROW, 128), jnp.float32),
        jax.ShapeDtypeStruct((NC, NROW, 128), jnp.float32),
    ),
    mesh=_mesh,
    scratch_types=[
        pltpu.VMEM((DNCH, CHUNK), jnp.int32),
        pltpu.VMEM((DNCH, CHUNK), jnp.int32),
        pltpu.VMEM((NROW, 128), jnp.float32),
        pltpu.VMEM((NROW, 128), jnp.float32),
        pltpu.VMEM((NS, 8, 128), jnp.float32),
        pltpu.VMEM((8, 128), jnp.float32),
        pltpu.VMEM_SHARED((NS, NROW, 128), jnp.float32),
        pltpu.VMEM_SHARED((NS, NROW, 128), jnp.float32),
    ],
    compiler_params=pltpu.CompilerParams(needs_layout_passes=False),
)(_deg_body)


NBUF = 2
WND = 64


def _scatter_body(t_hbm, srcr, dstr, out, src_v, dst_v, rows_v, acc_sh,
                  *sems):
    gsem = sems[:NBUF]
    ssem = sems[NBUF:]
    D = rows_v.shape[2]
    c = lax.axis_index("c")
    s = lax.axis_index("s")

    _zero_vmem(rows_v.at[0], CHUNK, D)
    base = pl.multiple_of(s * RPT, 8)
    off0 = 0
    while off0 < RPT:
        w0 = min(CHUNK, RPT - off0)
        pltpu.sync_copy(rows_v.at[0].at[pl.ds(0, w0)],
                        acc_sh.at[pl.ds(base + off0, w0)])
        off0 += w0
    plsc.subcore_barrier()

    def _wait_gather(b):
        pltpu.make_async_copy(
            t_hbm.at[src_v.at[0]], rows_v.at[b], gsem[b]).wait()

    def _wait_scatter(b):
        pltpu.make_async_copy(
            rows_v.at[b], acc_sh.at[dst_v.at[0]], ssem[b]).wait()

    def run(chunk0, nchunk):
        off = 0
        while off < nchunk:
            w = min(WND, nchunk - off)
            pltpu.sync_copy(srcr.at[s, pl.ds(chunk0 + off, w)],
                            src_v.at[pl.ds(0, w)])
            pltpu.sync_copy(dstr.at[s, pl.ds(chunk0 + off, w)],
                            dst_v.at[pl.ds(0, w)])
            for b in range(NBUF):
                pltpu.async_copy(t_hbm.at[src_v.at[b]], rows_v.at[b], gsem[b])

            def outer(g, _):
                j0 = g * NBUF
                for b in range(NBUF):
                    _wait_gather(b)
                    pltpu.async_copy(rows_v.at[b], acc_sh.at[dst_v.at[j0 + b]],
                                     ssem[b], add=True)
                for b in range(NBUF):
                    _wait_scatter(b)

                    @pl.when(j0 + NBUF + b < w)
                    def _():
                        pltpu.async_copy(t_hbm.at[src_v.at[j0 + NBUF + b]],
                                         rows_v.at[b], gsem[b])
                return 0
            lax.fori_loop(0, w // NBUF, outer, 0)
            off += w

    @pl.when(c == 0)
    def _():
        run(0, NCH0)

    @pl.when(c == 1)
    def _():
        run(NCH0, NCH1)

    plsc.subcore_barrier()
    pltpu.sync_copy(acc_sh.at[pl.ds(base, RPT)], out.at[c, pl.ds(base, RPT)])


def _make_scatter(D):
    return functools.partial(
        pl.kernel,
        out_type=jax.ShapeDtypeStruct((NC, NP, D), jnp.float32),
        mesh=_mesh,
        scratch_types=[
            pltpu.VMEM((WND, CHUNK), jnp.int32),
            pltpu.VMEM((WND, CHUNK), jnp.int32),
            pltpu.VMEM((NBUF, CHUNK, D), jnp.float32),
            pltpu.VMEM_SHARED((NP, D), jnp.float32),
        ] + [pltpu.SemaphoreType.DMA] * (2 * NBUF),
        compiler_params=pltpu.CompilerParams(needs_layout_passes=False),
    )(_scatter_body)


_scatter_128 = _make_scatter(128)


_RT = 1264
_RF = 1000


def _norm_from(deg2):
    d = deg2[0] + deg2[1]
    return jnp.where(d > 0, lax.rsqrt(jnp.maximum(d, 1.0)), 0.0)


def _tcmm_body(x_ref, w_ref, t_ref):
    t_ref[...] = jnp.dot(x_ref[...], w_ref[...],
                         preferred_element_type=jnp.float32)


def _tcscale_body(t_ref, dego_ref, o_ref):
    o_ref[...] = t_ref[...] * _norm_from(dego_ref[...])


def _tc_mid_body(acc_ref, dego_ref, degi_ref, b_ref, w_ref, t_ref):
    nd = _norm_from(degi_ref[...])
    agg = (acc_ref[0] + acc_ref[1]) * nd + b_ref[...]
    h = jnp.maximum(agg, 0.0)
    ns = _norm_from(dego_ref[...])
    t_ref[...] = jnp.dot(h * ns, w_ref[...], preferred_element_type=jnp.float32)


def _tc_fin_body(acc_ref, degi_ref, b_ref, o_ref):
    nd = _norm_from(degi_ref[...])
    agg = (acc_ref[0] + acc_ref[1])[:, :64] * nd + b_ref[...]
    o_ref[...] = jax.nn.sigmoid(agg) + 1e-8


def _deg_spec():
    return pl.BlockSpec((NC, _RT, 1), lambda i: (0, i, 0))


def _tcmm(x, w):
    dout = w.shape[1]
    return pl.pallas_call(
        _tcmm_body,
        grid=(N // _RF,),
        in_specs=[
            pl.BlockSpec((_RF, x.shape[1]), lambda i: (i, 0)),
            pl.BlockSpec(w.shape, lambda i: (0, 0)),
        ],
        out_specs=pl.BlockSpec((_RF, dout), lambda i: (i, 0)),
        out_shape=jax.ShapeDtypeStruct((NP, dout), jnp.float32),
    )(x, w)


def _tcscale(t, dego):
    dout = t.shape[1]
    return pl.pallas_call(
        _tcscale_body,
        grid=(NP // _RT,),
        in_specs=[
            pl.BlockSpec((_RT, dout), lambda i: (i, 0)),
            _deg_spec(),
        ],
        out_specs=pl.BlockSpec((_RT, dout), lambda i: (i, 0)),
        out_shape=jax.ShapeDtypeStruct((NP, dout), jnp.float32),
    )(t, dego)


def _tc_mid(acc, dego, degi, b, w):
    din, dout = w.shape
    return pl.pallas_call(
        _tc_mid_body,
        grid=(NP // _RT,),
        in_specs=[
            pl.BlockSpec((NC, _RT, din), lambda i: (0, i, 0)),
            _deg_spec(),
            _deg_spec(),
            pl.BlockSpec((1, din), lambda i: (0, 0)),
            pl.BlockSpec(w.shape, lambda i: (0, 0)),
        ],
        out_specs=pl.BlockSpec((_RT, dout), lambda i: (i, 0)),
        out_shape=jax.ShapeDtypeStruct((NP, dout), jnp.float32),
    )(acc, dego, degi, b, w)


def _tc_fin(acc, degi, b):
    return pl.pallas_call(
        _tc_fin_body,
        grid=(N // _RF,),
        in_specs=[
            pl.BlockSpec((NC, _RF, 128), lambda i: (0, i, 0)),
            pl.BlockSpec((NC, _RF, 1), lambda i: (0, i, 0)),
            pl.BlockSpec((1, 64), lambda i: (0, 0)),
        ],
        out_specs=pl.BlockSpec((_RF, 64), lambda i: (i, 0)),
        out_shape=jax.ShapeDtypeStruct((N, 64), jnp.float32),
    )(acc, degi, b)


def kernel(features, edge_index, W0, b0, W1, b1, W2, b2):
    pad = EP - E
    ep = jnp.concatenate(
        [edge_index, jnp.full((2, pad), N, dtype=jnp.int32)], axis=1)
    src = ep[0].reshape(NS, TOTCH, CHUNK)
    dst = ep[1].reshape(NS, TOTCH, CHUNK)
    src_dg = ep[0].reshape(NC, NS, DNCH, CHUNK)
    dst_dg = ep[1].reshape(NC, NS, DNCH, CHUNK)
    dego_g, degi_g = _deg_call(src_dg, dst_dg)
    dego = dego_g.reshape(NC, NPD, 1)
    degi = degi_g.reshape(NC, NPD, 1)
    t0r = _tcmm(features, W0)
    t0 = _tcscale(t0r, dego)
    a0 = _scatter_128(t0, src, dst)
    t1 = _tc_mid(a0, dego, degi, b0.reshape(1, -1), W1)
    a1 = _scatter_128(t1, src, dst)
    W2p = jnp.pad(W2, ((0, 0), (0, 64)))
    t2 = _tc_mid(a1, dego, degi, b1.reshape(1, -1), W2p)
    a2 = _scatter_128(t2, src, dst)
    return _tc_fin(a2, degi, b2.reshape(1, -1))


# speedup vs baseline: 1.0108x; 1.0008x over previous
import functools

import jax
import jax.numpy as jnp
from jax import lax
from jax.experimental import pallas as pl
from jax.experimental.pallas import tpu as pltpu
from jax.experimental.pallas import tpu_sc as plsc

N = 10000
NP = 10112
NPD = 10240
E = 320000
NC = 2
NS = 16
NW = NC * NS
CHUNK = 128
TOTCH = 158
NCH0 = 128
NCH1 = TOTCH - NCH0
EP = NS * TOTCH * CHUNK
DNCH = EP // (NW * CHUNK)
RPT = NP // NS
NROW = NPD // 128

_mesh = plsc.VectorSubcoreMesh(core_axis_name="c", subcore_axis_name="s")


def _zero_vmem(ref, nrow, ncol):
    def zrow(i, _):
        def zcol(j, _):
            ref[i, pl.ds(j * 16, 16)] = jnp.zeros((16,), jnp.float32)
            return 0
        return lax.fori_loop(0, ncol // 16, zcol, 0)
    lax.fori_loop(0, nrow, zrow, 0)


def _deg_body(srcr, dstr, dego, degi, src_v, dst_v, hist_o, hist_i,
              red_in, red_out, ho_sh, hi_sh):
    c = lax.axis_index("c")
    s = lax.axis_index("s")
    pltpu.sync_copy(srcr.at[c, s], src_v)
    pltpu.sync_copy(dstr.at[c, s], dst_v)
    _zero_vmem(hist_o, NROW, 128)
    _zero_vmem(hist_i, NROW, 128)
    ones = jnp.ones((16,), jnp.float32)

    def body(j, _):
        def inner(k, _):
            si = src_v[j, pl.ds(k * 16, 16)]
            plsc.addupdate_scatter(
                hist_o,
                [lax.shift_right_logical(si, 7), jnp.bitwise_and(si, 127)],
                ones)
            di = dst_v[j, pl.ds(k * 16, 16)]
            plsc.addupdate_scatter(
                hist_i,
                [lax.shift_right_logical(di, 7), jnp.bitwise_and(di, 127)],
                ones)
            return 0
        return lax.fori_loop(0, 8, inner, 0)
    lax.fori_loop(0, DNCH, body, 0)

    pltpu.sync_copy(hist_o, ho_sh.at[s])
    pltpu.sync_copy(hist_i, hi_sh.at[s])
    plsc.subcore_barrier()

    @pl.when(s < NROW // 8)
    def _():
        rb = pl.multiple_of(8 * s, 8)

        def _reduce(sh, out_hbm):
            for r in range(NS):
                pltpu.sync_copy(sh.at[r, pl.ds(rb, 8)], red_in.at[r])

            def ri(i, _):
                def rj(jj, _):
                    acc = red_in[0, i, pl.ds(jj * 16, 16)]
                    for r in range(1, NS):
                        acc = acc + red_in[r, i, pl.ds(jj * 16, 16)]
                    red_out[i, pl.ds(jj * 16, 16)] = acc
                    return 0
                return lax.fori_loop(0, 8, rj, 0)
            lax.fori_loop(0, 8, ri, 0)
            pltpu.sync_copy(red_out, out_hbm.at[c, pl.ds(rb, 8)])

        _reduce(ho_sh, dego)
        _reduce(hi_sh, degi)


_deg_call = functools.partial(
    pl.kernel,
    out_type=(
        jax.ShapeDtypeStruct((NC, NROW, 128), jnp.float32),
        jax.ShapeDtypeStruct((NC, NROW, 128), jnp.float32),
    ),
    mesh=_mesh,
    scratch_types=[
        pltpu.VMEM((DNCH, CHUNK), jnp.int32),
        pltpu.VMEM((DNCH, CHUNK), jnp.int32),
        pltpu.VMEM((NROW, 128), jnp.float32),
        pltpu.VMEM((NROW, 128), jnp.float32),
        pltpu.VMEM((NS, 8, 128), jnp.float32),
        pltpu.VMEM((8, 128), jnp.float32),
        pltpu.VMEM_SHARED((NS, NROW, 128), jnp.float32),
        pltpu.VMEM_SHARED((NS, NROW, 128), jnp.float32),
    ],
    compiler_params=pltpu.CompilerParams(needs_layout_passes=False),
)(_deg_body)


NBUF = 2
WND = 64


def _scatter_body(t_hbm, srcr, dstr, out, src_v, dst_v, rows_v, acc_sh,
                  *sems):
    gsem = sems[:NBUF]
    ssem = sems[NBUF:]
    D = rows_v.shape[2]
    c = lax.axis_index("c")
    s = lax.axis_index("s")

    _zero_vmem(rows_v.at[0], CHUNK, D)
    base = pl.multiple_of(s * RPT, 8)
    off0 = 0
    while off0 < RPT:
        w0 = min(CHUNK, RPT - off0)
        pltpu.sync_copy(rows_v.at[0].at[pl.ds(0, w0)],
                        acc_sh.at[pl.ds(base + off0, w0)])
        off0 += w0
    plsc.subcore_barrier()

    def _wait_gather(b):
        pltpu.make_async_copy(
            t_hbm.at[src_v.at[0]], rows_v.at[b], gsem[b]).wait()

    def _wait_scatter(b):
        pltpu.make_async_copy(
            rows_v.at[b], acc_sh.at[dst_v.at[0]], ssem[b]).wait()

    def run(chunk0, nchunk):
        off = 0
        while off < nchunk:
            w = min(WND, nchunk - off)
            pltpu.sync_copy(srcr.at[s, pl.ds(chunk0 + off, w)],
                            src_v.at[pl.ds(0, w)])
            pltpu.sync_copy(dstr.at[s, pl.ds(chunk0 + off, w)],
                            dst_v.at[pl.ds(0, w)])
            for b in range(NBUF):
                pltpu.async_copy(t_hbm.at[src_v.at[b]], rows_v.at[b], gsem[b])

            def outer(g, _):
                j0 = g * NBUF
                for b in range(NBUF):
                    _wait_gather(b)
                    pltpu.async_copy(rows_v.at[b], acc_sh.at[dst_v.at[j0 + b]],
                                     ssem[b], add=True)
                for b in range(NBUF):
                    _wait_scatter(b)

                    @pl.when(j0 + NBUF + b < w)
                    def _():
                        pltpu.async_copy(t_hbm.at[src_v.at[j0 + NBUF + b]],
                                         rows_v.at[b], gsem[b])
                return 0
            lax.fori_loop(0, w // NBUF, outer, 0, unroll=2)
            off += w

    @pl.when(c == 0)
    def _():
        run(0, NCH0)

    @pl.when(c == 1)
    def _():
        run(NCH0, NCH1)

    plsc.subcore_barrier()
    pltpu.sync_copy(acc_sh.at[pl.ds(base, RPT)], out.at[c, pl.ds(base, RPT)])


def _make_scatter(D):
    return functools.partial(
        pl.kernel,
        out_type=jax.ShapeDtypeStruct((NC, NP, D), jnp.float32),
        mesh=_mesh,
        scratch_types=[
            pltpu.VMEM((WND, CHUNK), jnp.int32),
            pltpu.VMEM((WND, CHUNK), jnp.int32),
            pltpu.VMEM((NBUF, CHUNK, D), jnp.float32),
            pltpu.VMEM_SHARED((NP, D), jnp.float32),
        ] + [pltpu.SemaphoreType.DMA] * (2 * NBUF),
        compiler_params=pltpu.CompilerParams(needs_layout_passes=False),
    )(_scatter_body)


_scatter_128 = _make_scatter(128)


_RT = 1264
_RF = 1000


def _norm_from(deg2):
    d = deg2[0] + deg2[1]
    return jnp.where(d > 0, lax.rsqrt(jnp.maximum(d, 1.0)), 0.0)


def _tcmm_body(x_ref, w_ref, t_ref):
    t_ref[...] = jnp.dot(x_ref[...], w_ref[...],
                         preferred_element_type=jnp.float32)


def _tcscale_body(t_ref, dego_ref, o_ref):
    o_ref[...] = t_ref[...] * _norm_from(dego_ref[...])


def _tc_mid_body(acc_ref, dego_ref, degi_ref, b_ref, w_ref, t_ref):
    nd = _norm_from(degi_ref[...])
    agg = (acc_ref[0] + acc_ref[1]) * nd + b_ref[...]
    h = jnp.maximum(agg, 0.0)
    ns = _norm_from(dego_ref[...])
    t_ref[...] = jnp.dot(h * ns, w_ref[...], preferred_element_type=jnp.float32)


def _tc_fin_body(acc_ref, degi_ref, b_ref, o_ref):
    nd = _norm_from(degi_ref[...])
    agg = (acc_ref[0] + acc_ref[1])[:, :64] * nd + b_ref[...]
    o_ref[...] = jax.nn.sigmoid(agg) + 1e-8


def _deg_spec():
    return pl.BlockSpec((NC, _RT, 1), lambda i: (0, i, 0))


def _tcmm(x, w):
    dout = w.shape[1]
    return pl.pallas_call(
        _tcmm_body,
        grid=(N // _RF,),
        in_specs=[
            pl.BlockSpec((_RF, x.shape[1]), lambda i: (i, 0)),
            pl.BlockSpec(w.shape, lambda i: (0, 0)),
        ],
        out_specs=pl.BlockSpec((_RF, dout), lambda i: (i, 0)),
        out_shape=jax.ShapeDtypeStruct((NP, dout), jnp.float32),
    )(x, w)


def _tcscale(t, dego):
    dout = t.shape[1]
    return pl.pallas_call(
        _tcscale_body,
        grid=(NP // _RT,),
        in_specs=[
            pl.BlockSpec((_RT, dout), lambda i: (i, 0)),
            _deg_spec(),
        ],
        out_specs=pl.BlockSpec((_RT, dout), lambda i: (i, 0)),
        out_shape=jax.ShapeDtypeStruct((NP, dout), jnp.float32),
    )(t, dego)


def _tc_mid(acc, dego, degi, b, w):
    din, dout = w.shape
    return pl.pallas_call(
        _tc_mid_body,
        grid=(NP // _RT,),
        in_specs=[
            pl.BlockSpec((NC, _RT, din), lambda i: (0, i, 0)),
            _deg_spec(),
            _deg_spec(),
            pl.BlockSpec((1, din), lambda i: (0, 0)),
            pl.BlockSpec(w.shape, lambda i: (0, 0)),
        ],
        out_specs=pl.BlockSpec((_RT, dout), lambda i: (i, 0)),
        out_shape=jax.ShapeDtypeStruct((NP, dout), jnp.float32),
    )(acc, dego, degi, b, w)


def _tc_fin(acc, degi, b):
    return pl.pallas_call(
        _tc_fin_body,
        grid=(N // _RF,),
        in_specs=[
            pl.BlockSpec((NC, _RF, 128), lambda i: (0, i, 0)),
            pl.BlockSpec((NC, _RF, 1), lambda i: (0, i, 0)),
            pl.BlockSpec((1, 64), lambda i: (0, 0)),
        ],
        out_specs=pl.BlockSpec((_RF, 64), lambda i: (i, 0)),
        out_shape=jax.ShapeDtypeStruct((N, 64), jnp.float32),
    )(acc, degi, b)


def kernel(features, edge_index, W0, b0, W1, b1, W2, b2):
    pad = EP - E
    ep = jnp.concatenate(
        [edge_index, jnp.full((2, pad), N, dtype=jnp.int32)], axis=1)
    src = ep[0].reshape(NS, TOTCH, CHUNK)
    dst = ep[1].reshape(NS, TOTCH, CHUNK)
    src_dg = ep[0].reshape(NC, NS, DNCH, CHUNK)
    dst_dg = ep[1].reshape(NC, NS, DNCH, CHUNK)
    dego_g, degi_g = _deg_call(src_dg, dst_dg)
    dego = dego_g.reshape(NC, NPD, 1)
    degi = degi_g.reshape(NC, NPD, 1)
    t0r = _tcmm(features, W0)
    t0 = _tcscale(t0r, dego)
    a0 = _scatter_128(t0, src, dst)
    t1 = _tc_mid(a0, dego, degi, b0.reshape(1, -1), W1)
    a1 = _scatter_128(t1, src, dst)
    W2p = jnp.pad(W2, ((0, 0), (0, 64)))
    t2 = _tc_mid(a1, dego, degi, b1.reshape(1, -1), W2p)
    a2 = _scatter_128(t2, src, dst)
    return _tc_fin(a2, degi, b2.reshape(1, -1))
